# Initial kernel scaffold; baseline (speedup 1.0000x reference)
import jax
import jax.numpy as jnp
from jax.experimental import pallas as pl


def kernel(x, table, W, b):
    raise NotImplementedError("write your pallas kernel here")


# speedup vs baseline: 17.9343x; 17.9343x over previous
import functools

import jax
import jax.numpy as jnp
from jax import lax
from jax.experimental import pallas as pl
from jax.experimental.pallas import tpu as pltpu
from jax.experimental.pallas import tpu_sc as plsc

NC = 2
NS = 16
NW = NC * NS
L = 16

B = 16384
S = 50
D = 32
CB = B // NW
NPAIR = CB // 2
G = 2 * S
NBUF = 4
NOUTER = NPAIR // NBUF


def _tree_sum(vals):
    accs = list(vals[:4])
    for i in range(4, len(vals)):
        accs[i % 4] = accs[i % 4] + vals[i]
    return (accs[0] + accs[1]) + (accs[2] + accs[3])


def _sc_body(idx_hbm, props_hbm, table_hbm, w_hbm, b_hbm, out_hbm,
             idx_v, props_v, ps_v, out_v, bufs, wv, bv, sems):
    wid = lax.axis_index("s") * NC + lax.axis_index("c")

    pltpu.sync_copy(idx_hbm.at[wid], idx_v)
    pltpu.sync_copy(props_hbm.at[wid], props_v)
    pltpu.sync_copy(w_hbm, wv)
    pltpu.sync_copy(b_hbm, bv)

    w_h = [wv[pl.ds(0, L)], wv[pl.ds(L, L)]]
    sb_h = [bv[pl.ds(0, L)] * float(S), bv[pl.ds(L, L)] * float(S)]

    def ps_body(c, carry):
        base = c * L
        ps_v[pl.ds(base, L)] = _tree_sum(
            [props_v[s, pl.ds(base, L)] for s in range(S)])
        return carry

    lax.fori_loop(0, CB // L, ps_body, 0)

    def start(j, buf, sem):
        pltpu.make_async_copy(table_hbm.at[idx_v.at[j]], buf, sem).start()

    def accum(j, buf):
        ps_vec = ps_v[pl.ds(2 * j, L)]
        for r in range(2):
            row = 2 * j + r
            ps_s = ps_vec[r]
            for h in range(2):
                tot = _tree_sum(
                    [buf[r * S + i, pl.ds(h * L, L)] for i in range(S)])
                out_v[row, pl.ds(h * L, L)] = tot + ps_s * w_h[h] + sb_h[h]

    for bi in range(NBUF):
        start(bi, bufs[bi], sems[bi])

    def body(i, carry):
        for bi in range(NBUF):
            j = i * NBUF + bi
            pltpu.make_async_copy(
                table_hbm.at[idx_v.at[j]], bufs[bi], sems[bi]).wait()
            accum(j, bufs[bi])

            @pl.when(i < NOUTER - 1)
            def _():
                start(j + NBUF, bufs[bi], sems[bi])
        return carry

    lax.fori_loop(0, NOUTER, body, 0)

    pltpu.sync_copy(out_v, out_hbm.at[pl.ds(wid * CB, CB)])


@functools.lru_cache(maxsize=1)
def _make_sc_kernel():
    @functools.partial(
        pl.kernel,
        out_type=jax.ShapeDtypeStruct((B, D), jnp.float32),
        mesh=plsc.VectorSubcoreMesh(core_axis_name="c", subcore_axis_name="s",
                                    num_cores=NC, num_subcores=NS),
        compiler_params=pltpu.CompilerParams(use_tc_tiling_on_sc=False),
        scratch_types=dict(
            idx_v=pltpu.VMEM((NPAIR, G), jnp.int32),
            props_v=pltpu.VMEM((S, CB), jnp.float32),
            ps_v=pltpu.VMEM((CB + L,), jnp.float32),
            out_v=pltpu.VMEM((CB, D), jnp.float32),
            bufs=[pltpu.VMEM((G, D), jnp.float32) for _ in range(NBUF)],
            wv=pltpu.VMEM((D,), jnp.float32),
            bv=pltpu.VMEM((D,), jnp.float32),
            sems=[pltpu.SemaphoreType.DMA for _ in range(NBUF)],
        ),
    )
    def _sc_kernel(idx_hbm, props_hbm, table_hbm, w_hbm, b_hbm, out_hbm,
                   idx_v, props_v, ps_v, out_v, bufs, wv, bv, sems):
        _sc_body(idx_hbm, props_hbm, table_hbm, w_hbm, b_hbm, out_hbm,
                 idx_v, props_v, ps_v, out_v, bufs, wv, bv, sems)

    return _sc_kernel


def kernel(x, table, W, b):
    idx = x[..., 0].astype(jnp.int32).reshape(NW, NPAIR, G)
    props = x[..., 1].reshape(NW, CB, S).transpose(0, 2, 1)
    w = W[:, 0]
    return _make_sc_kernel()(idx, props, table, w, b)
